# baseline (speedup 1.0000x reference)
import jax
import jax.numpy as jnp
from jax.experimental import pallas as pl

_E = 8
_K = 2
_TAU = 1.0
_TOKENS = 4096
_D = 1024
_BLK = 128


def _gate_combine_kernel(x_ref, gwt_ref, bias_ref, h_ref, y_ref, gst_ref, pm_ref):
    f32 = jnp.float32
    T = x_ref.shape[0]

    logits = jnp.dot(x_ref[...], gwt_ref[...], preferred_element_type=f32)
    logits = (logits + bias_ref[...]) * f32(1.0 / _TAU)
    lt = jnp.transpose(logits)

    row = jax.lax.broadcasted_iota(jnp.int32, (_E, T), 0).astype(f32)

    m = jnp.max(lt, axis=0, keepdims=True)
    p = jnp.exp(lt - m)
    g = p / jnp.sum(p, axis=0, keepdims=True)

    i1 = jnp.min(jnp.where(lt == m, row, f32(_E)), axis=0, keepdims=True)
    l2 = jnp.where(row == i1, f32(-jnp.inf), lt)
    m2 = jnp.max(l2, axis=0, keepdims=True)
    i2 = jnp.min(jnp.where(l2 == m2, row, f32(_E)), axis=0, keepdims=True)
    mask = jnp.logical_or(row == i1, row == i2)

    adjusted = jnp.where(mask, lt - jnp.log(f32(_K) * (g + f32(1e-10))), f32(-1e9))
    am = jnp.max(adjusted, axis=0, keepdims=True)
    ap = jnp.exp(adjusted - am)
    gst = ap / jnp.sum(ap, axis=0, keepdims=True)
    gst_ref[...] = gst

    rcol = jax.lax.broadcasted_iota(jnp.int32, (_E, 1), 0).astype(f32)
    pm = jnp.zeros((_E, 1), f32)
    gcur = g
    for r in range(_E):
        mr = jnp.max(gcur, axis=0, keepdims=True)
        pm = pm + jnp.sum(mr) * jnp.where(rcol == f32(r), f32(1.0), f32(0.0))
        ir = jnp.min(jnp.where(gcur == mr, row, f32(_E)), axis=0, keepdims=True)
        gcur = jnp.where(row == ir, f32(-1.0), gcur)

    @pl.when(pl.program_id(0) == 0)
    def _init():
        pm_ref[...] = jnp.zeros_like(pm_ref)

    pm_ref[...] += pm

    @pl.when(pl.program_id(0) == pl.num_programs(0) - 1)
    def _finish():
        pm_ref[...] = pm_ref[...] * f32(1.0 / _TOKENS)

    gstt = jnp.transpose(gst)
    tt = jax.lax.broadcasted_iota(jnp.int32, (_E, _E * _E), 0)
    cc = jax.lax.broadcasted_iota(jnp.int32, (_E, _E * _E), 1)
    diag = jnp.where(cc // _E == tt, f32(1.0), f32(0.0))
    for i in range(T // _E):
        gsub = gstt[i * _E:(i + 1) * _E, :]
        bsub = jnp.concatenate([gsub] * _E, axis=1) * diag
        y_ref[i * _E:(i + 1) * _E, :] = jnp.dot(
            bsub, h_ref[i * _E * _E:(i + 1) * _E * _E, :],
            preferred_element_type=f32)


@jax.jit
def _run(h, x, gwt, bias2):
    ht = jnp.transpose(h, (0, 2, 1)).reshape(_TOKENS * _E, _D)
    grid = (_TOKENS // _BLK,)
    y, gst, pm = pl.pallas_call(
        _gate_combine_kernel,
        grid=grid,
        in_specs=[
            pl.BlockSpec((_BLK, _D), lambda i: (i, 0)),
            pl.BlockSpec((_D, _E), lambda i: (0, 0)),
            pl.BlockSpec((1, _E), lambda i: (0, 0)),
            pl.BlockSpec((_BLK * _E, _D), lambda i: (i, 0)),
        ],
        out_specs=[
            pl.BlockSpec((_BLK, _D), lambda i: (i, 0)),
            pl.BlockSpec((_E, _BLK), lambda i: (0, i)),
            pl.BlockSpec((_E, 1), lambda i: (0, 0)),
        ],
        out_shape=[
            jax.ShapeDtypeStruct((_TOKENS, _D), jnp.float32),
            jax.ShapeDtypeStruct((_E, _TOKENS), jnp.float32),
            jax.ShapeDtypeStruct((_E, 1), jnp.float32),
        ],
    )(x, gwt, bias2, ht)
    return y, jnp.transpose(gst), pm.reshape(_E)


def kernel(h, x, permutation_weights, gate_weights, bias):
    del permutation_weights
    return _run(h, x, gate_weights.T, bias.reshape(1, _E))


# speedup vs baseline: 1.1620x; 1.1620x over previous
import jax
import jax.numpy as jnp
from jax.experimental import pallas as pl

_E = 8
_K = 2
_TAU = 1.0
_TOKENS = 4096
_D = 1024
_BLK = 256


def _gate_combine_kernel(x_ref, gwt_ref, bias_ref, h_ref, y_ref, gst_ref, pm_ref):
    f32 = jnp.float32
    T = x_ref.shape[0]

    logits = jnp.dot(x_ref[...], gwt_ref[...], preferred_element_type=f32)
    logits = (logits + bias_ref[...]) * f32(1.0 / _TAU)
    lt = jnp.transpose(logits)

    row = jax.lax.broadcasted_iota(jnp.int32, (_E, T), 0).astype(f32)

    m = jnp.max(lt, axis=0, keepdims=True)
    p = jnp.exp(lt - m)
    g = p / jnp.sum(p, axis=0, keepdims=True)

    i1 = jnp.min(jnp.where(lt == m, row, f32(_E)), axis=0, keepdims=True)
    l2 = jnp.where(row == i1, f32(-jnp.inf), lt)
    m2 = jnp.max(l2, axis=0, keepdims=True)
    i2 = jnp.min(jnp.where(l2 == m2, row, f32(_E)), axis=0, keepdims=True)
    mask = jnp.logical_or(row == i1, row == i2)

    adjusted = jnp.where(mask, lt - jnp.log(f32(_K) * (g + f32(1e-10))), f32(-1e9))
    am = jnp.max(adjusted, axis=0, keepdims=True)
    ap = jnp.exp(adjusted - am)
    gst = ap / jnp.sum(ap, axis=0, keepdims=True)
    gst_ref[...] = gst

    rcol = jax.lax.broadcasted_iota(jnp.int32, (_E, 1), 0).astype(f32)
    pm = jnp.zeros((_E, 1), f32)
    gcur = g
    for r in range(_E):
        mr = jnp.max(gcur, axis=0, keepdims=True)
        pm = pm + jnp.sum(mr) * jnp.where(rcol == f32(r), f32(1.0), f32(0.0))
        ir = jnp.min(jnp.where(gcur == mr, row, f32(_E)), axis=0, keepdims=True)
        gcur = jnp.where(row == ir, f32(-1.0), gcur)

    @pl.when(pl.program_id(0) == 0)
    def _init():
        pm_ref[...] = jnp.zeros_like(pm_ref)

    pm_ref[...] += pm

    @pl.when(pl.program_id(0) == pl.num_programs(0) - 1)
    def _finish():
        pm_ref[...] = pm_ref[...] * f32(1.0 / _TOKENS)

    gstt = jnp.transpose(gst)
    tt = jax.lax.broadcasted_iota(jnp.int32, (_E, _E * _E), 0)
    cc = jax.lax.broadcasted_iota(jnp.int32, (_E, _E * _E), 1)
    diag = jnp.where(cc // _E == tt, f32(1.0), f32(0.0))
    for i in range(T // _E):
        gsub = gstt[i * _E:(i + 1) * _E, :]
        bsub = jnp.concatenate([gsub] * _E, axis=1) * diag
        y_ref[i * _E:(i + 1) * _E, :] = jnp.dot(
            bsub, h_ref[i * _E * _E:(i + 1) * _E * _E, :],
            preferred_element_type=f32)


@jax.jit
def _run(h, x, gwt, bias2):
    ht = jnp.transpose(h, (0, 2, 1)).reshape(_TOKENS * _E, _D)
    grid = (_TOKENS // _BLK,)
    y, gst, pm = pl.pallas_call(
        _gate_combine_kernel,
        grid=grid,
        in_specs=[
            pl.BlockSpec((_BLK, _D), lambda i: (i, 0)),
            pl.BlockSpec((_D, _E), lambda i: (0, 0)),
            pl.BlockSpec((1, _E), lambda i: (0, 0)),
            pl.BlockSpec((_BLK * _E, _D), lambda i: (i, 0)),
        ],
        out_specs=[
            pl.BlockSpec((_BLK, _D), lambda i: (i, 0)),
            pl.BlockSpec((_E, _BLK), lambda i: (0, i)),
            pl.BlockSpec((_E, 1), lambda i: (0, 0)),
        ],
        out_shape=[
            jax.ShapeDtypeStruct((_TOKENS, _D), jnp.float32),
            jax.ShapeDtypeStruct((_E, _TOKENS), jnp.float32),
            jax.ShapeDtypeStruct((_E, 1), jnp.float32),
        ],
    )(x, gwt, bias2, ht)
    return y, jnp.transpose(gst), pm.reshape(_E)


def kernel(h, x, permutation_weights, gate_weights, bias):
    del permutation_weights
    return _run(h, x, gate_weights.T, bias.reshape(1, _E))
